# Initial kernel scaffold; baseline (speedup 1.0000x reference)
import jax
import jax.numpy as jnp
from jax.experimental import pallas as pl


def kernel(x, adj_list, W1, b1, W2, b2):
    raise NotImplementedError("write your pallas kernel here")


# speedup vs baseline: 1.1371x; 1.1371x over previous
import functools

import jax
import jax.numpy as jnp
from jax.experimental import pallas as pl
from jax.experimental.pallas import tpu as pltpu

_LAMBD0 = 0.0051
_LAMBD1 = 0.0051
_LAMBD2 = 0.0051
_W_LOSS1 = 1.0
_W_LOSS2 = 1.0

_N = 10000
_F = 128
_BM = 200
_NBLK = _N // _BM


def _bt_loss(cm, lam):
    eye = (
        jax.lax.broadcasted_iota(jnp.int32, (_F, _F), 0)
        == jax.lax.broadcasted_iota(jnp.int32, (_F, _F), 1)
    ).astype(jnp.float32)
    total_sq = jnp.sum(cm * cm)
    diag = cm * eye
    diag_sq = jnp.sum(diag * diag)
    trace = jnp.sum(diag)
    on_diag = diag_sq - 2.0 * trace + float(_F)
    off_diag = total_sq - diag_sq
    return on_diag + lam * off_diag


def _body(adj_ref, x_ref, w1_ref, b1_ref, w2_ref, b2_ref, out_ref,
          ha_ref, ha1_ref, c_ref, c0_ref, c1_ref):
    i = pl.program_id(0)

    @pl.when(i == 0)
    def _init():
        xv = x_ref[...]
        dn = (((1,), (1,)), ((), ()))
        ha_ref[...] = (
            jax.lax.dot_general(xv, w1_ref[...], dn,
                                preferred_element_type=jnp.float32)
            + b1_ref[...]
        )
        ha1_ref[...] = (
            jax.lax.dot_general(xv, w2_ref[...], dn,
                                preferred_element_type=jnp.float32)
            + b2_ref[...]
        )
        zeros = jnp.zeros((_F, _F), jnp.float32)
        c_ref[...] = zeros
        c0_ref[...] = zeros
        c1_ref[...] = zeros

    ha = ha_ref[...]
    hp0 = jnp.dot(adj_ref[0], ha, preferred_element_type=jnp.float32)
    hp1 = jnp.dot(adj_ref[1], ha, preferred_element_type=jnp.float32)

    ha_blk = ha_ref[pl.ds(i * _BM, _BM), :]
    ha1_blk = ha1_ref[pl.ds(i * _BM, _BM), :]

    dt = (((0,), (0,)), ((), ()))
    c_ref[...] += jax.lax.dot_general(hp1, hp0, dt,
                                      preferred_element_type=jnp.float32)
    c0_ref[...] += jax.lax.dot_general(hp0, ha_blk, dt,
                                       preferred_element_type=jnp.float32)
    c1_ref[...] += jax.lax.dot_general(hp1, ha1_blk, dt,
                                       preferred_element_type=jnp.float32)

    @pl.when(i == _NBLK - 1)
    def _finish():
        loss = (
            _bt_loss(c_ref[...], _LAMBD0)
            + _W_LOSS1 * _bt_loss(c0_ref[...], _LAMBD1)
            + _W_LOSS2 * _bt_loss(c1_ref[...], _LAMBD2)
        )
        out_ref[...] = jnp.reshape(loss, (1, 1))


@functools.partial(jax.jit, static_argnames=("interpret",))
def _run(x, adj_list, W1, b1, W2, b2, interpret=False):
    out = pl.pallas_call(
        _body,
        grid=(_NBLK,),
        in_specs=[
            pl.BlockSpec((2, _BM, _N), lambda i: (0, i, 0)),
            pl.BlockSpec((_N, _F), lambda i: (0, 0)),
            pl.BlockSpec((_F, _F), lambda i: (0, 0)),
            pl.BlockSpec((1, _F), lambda i: (0, 0)),
            pl.BlockSpec((_F, _F), lambda i: (0, 0)),
            pl.BlockSpec((1, _F), lambda i: (0, 0)),
        ],
        out_specs=pl.BlockSpec((1, 1), lambda i: (0, 0)),
        out_shape=jax.ShapeDtypeStruct((1, 1), jnp.float32),
        scratch_shapes=[
            pltpu.VMEM((_N, _F), jnp.float32),
            pltpu.VMEM((_N, _F), jnp.float32),
            pltpu.VMEM((_F, _F), jnp.float32),
            pltpu.VMEM((_F, _F), jnp.float32),
            pltpu.VMEM((_F, _F), jnp.float32),
        ],
        interpret=interpret,
    )(adj_list, x, W1, b1.reshape(1, _F), W2, b2.reshape(1, _F))
    return out[0, 0]


def kernel(x, adj_list, W1, b1, W2, b2):
    return _run(x, adj_list, W1, b1, W2, b2)


# speedup vs baseline: 1.1448x; 1.0067x over previous
import functools

import jax
import jax.numpy as jnp
from jax.experimental import pallas as pl
from jax.experimental.pallas import tpu as pltpu

_LAMBD0 = 0.0051
_LAMBD1 = 0.0051
_LAMBD2 = 0.0051
_W_LOSS1 = 1.0
_W_LOSS2 = 1.0

_N = 10000
_F = 128
_BM = 200
_NBLK = _N // _BM


def _bt_loss(cm, lam):
    eye = (
        jax.lax.broadcasted_iota(jnp.int32, (_F, _F), 0)
        == jax.lax.broadcasted_iota(jnp.int32, (_F, _F), 1)
    ).astype(jnp.float32)
    total_sq = jnp.sum(cm * cm)
    diag = cm * eye
    diag_sq = jnp.sum(diag * diag)
    trace = jnp.sum(diag)
    on_diag = diag_sq - 2.0 * trace + float(_F)
    off_diag = total_sq - diag_sq
    return on_diag + lam * off_diag


def _body(adj_ref, x_ref, w1_ref, b1_ref, w2_ref, b2_ref, out_ref,
          ha_ref, ha1_ref, c_ref, c0_ref, c1_ref):
    i = pl.program_id(0)

    @pl.when(i == 0)
    def _init():
        xv = x_ref[...]
        dn = (((1,), (1,)), ((), ()))
        ha_ref[...] = (
            jax.lax.dot_general(xv, w1_ref[...], dn,
                                preferred_element_type=jnp.float32)
            + b1_ref[...]
        )
        ha1_ref[...] = (
            jax.lax.dot_general(xv, w2_ref[...], dn,
                                preferred_element_type=jnp.float32)
            + b2_ref[...]
        )
        zeros = jnp.zeros((_F, _F), jnp.float32)
        c_ref[...] = zeros
        c0_ref[...] = zeros
        c1_ref[...] = zeros

    ha = ha_ref[...]
    a_both = adj_ref[...].reshape(2 * _BM, _N)
    hp_both = jnp.dot(a_both, ha, preferred_element_type=jnp.float32)
    hp0 = hp_both[:_BM]
    hp1 = hp_both[_BM:]

    ha_blk = ha_ref[pl.ds(i * _BM, _BM), :]
    ha1_blk = ha1_ref[pl.ds(i * _BM, _BM), :]

    dt = (((0,), (0,)), ((), ()))
    c_ref[...] += jax.lax.dot_general(hp1, hp0, dt,
                                      preferred_element_type=jnp.float32)
    c0_ref[...] += jax.lax.dot_general(hp0, ha_blk, dt,
                                       preferred_element_type=jnp.float32)
    c1_ref[...] += jax.lax.dot_general(hp1, ha1_blk, dt,
                                       preferred_element_type=jnp.float32)

    @pl.when(i == _NBLK - 1)
    def _finish():
        loss = (
            _bt_loss(c_ref[...], _LAMBD0)
            + _W_LOSS1 * _bt_loss(c0_ref[...], _LAMBD1)
            + _W_LOSS2 * _bt_loss(c1_ref[...], _LAMBD2)
        )
        out_ref[...] = jnp.reshape(loss, (1, 1))


@functools.partial(jax.jit, static_argnames=("interpret",))
def _run(x, adj_list, W1, b1, W2, b2, interpret=False):
    out = pl.pallas_call(
        _body,
        grid=(_NBLK,),
        in_specs=[
            pl.BlockSpec((2, _BM, _N), lambda i: (0, i, 0)),
            pl.BlockSpec((_N, _F), lambda i: (0, 0)),
            pl.BlockSpec((_F, _F), lambda i: (0, 0)),
            pl.BlockSpec((1, _F), lambda i: (0, 0)),
            pl.BlockSpec((_F, _F), lambda i: (0, 0)),
            pl.BlockSpec((1, _F), lambda i: (0, 0)),
        ],
        out_specs=pl.BlockSpec((1, 1), lambda i: (0, 0)),
        out_shape=jax.ShapeDtypeStruct((1, 1), jnp.float32),
        scratch_shapes=[
            pltpu.VMEM((_N, _F), jnp.float32),
            pltpu.VMEM((_N, _F), jnp.float32),
            pltpu.VMEM((_F, _F), jnp.float32),
            pltpu.VMEM((_F, _F), jnp.float32),
            pltpu.VMEM((_F, _F), jnp.float32),
        ],
        interpret=interpret,
        compiler_params=pltpu.CompilerParams(
            vmem_limit_bytes=100 * 1024 * 1024,
        ),
    )(adj_list, x, W1, b1.reshape(1, _F), W2, b2.reshape(1, _F))
    return out[0, 0]


def kernel(x, adj_list, W1, b1, W2, b2):
    return _run(x, adj_list, W1, b1, W2, b2)
